# baseline (speedup 1.0000x reference)
import functools
import jax
import jax.numpy as jnp
from jax import lax
from jax.experimental import pallas as pl
from jax.experimental.pallas import tpu as pltpu
from jax.experimental.pallas import tpu_sc as plsc

N = 10000; E = 320000; B = 64; FEAT = 128; SUB = 256; MOL = 512; NF = 100000
NH1 = 5; H1 = 128; NH2 = 1; O2 = 256; SLOPE = 0.1

NC = 2
NS = 16
NW = NC * NS
L = 16
NPAD = 10240
EK = 80
E_PER_SC = E // NC
E_PER_TILE = E // NW
N_CHUNK = E_PER_TILE // EK
ROWS_PER_TILE = NPAD // NS

_MESH = plsc.VectorSubcoreMesh(core_axis_name="c", subcore_axis_name="s")


def _sc_gather(table, idx):
    V, D = table.shape
    n = idx.shape[0]
    per_w = n // NW
    n_chunk = per_w // EK

    @functools.partial(
        pl.kernel,
        out_type=jax.ShapeDtypeStruct((n, D), jnp.float32),
        mesh=_MESH,
        scratch_types=[
            pltpu.VMEM((EK,), jnp.int32),
            pltpu.VMEM((EK, D), jnp.float32),
            pltpu.SemaphoreType.DMA,
        ],
    )
    def k(table_hbm, idx_hbm, out_hbm, idx_v, rows_v, sem):
        wid = lax.axis_index("s") * NC + lax.axis_index("c")
        base = wid * per_w

        def chunk(i, _):
            off = base + i * EK
            pltpu.sync_copy(idx_hbm.at[pl.ds(off, EK)], idx_v)
            pltpu.async_copy(table_hbm.at[idx_v], rows_v, sem).wait()
            pltpu.sync_copy(rows_v, out_hbm.at[pl.ds(off, EK)])
            return 0

        lax.fori_loop(0, n_chunk, chunk, 0)

    return k(table, idx)


EKG = 40
N_CHUNK_G = E_PER_TILE // EKG


def _sc_edge_gate(tcomb, src, dst, zeros128):
    @functools.partial(
        pl.kernel,
        out_type=(
            jax.ShapeDtypeStruct((E * L,), jnp.float32),
            jax.ShapeDtypeStruct((NC, NPAD, 128), jnp.float32),
        ),
        mesh=_MESH,
        scratch_types=[
            pltpu.VMEM((E_PER_TILE,), jnp.int32),
            pltpu.VMEM((E_PER_TILE,), jnp.int32),
            pltpu.VMEM((EKG,), jnp.int32),
            pltpu.VMEM((EKG,), jnp.int32),
            pltpu.VMEM((EKG, 128), jnp.float32),
            pltpu.VMEM((EKG, 128), jnp.float32),
            pltpu.VMEM((EKG, 128), jnp.float32),
            pltpu.VMEM((EKG, 128), jnp.float32),
            pltpu.VMEM((EKG * L,), jnp.float32),
            pltpu.VMEM((EKG, 128), jnp.float32),
            pltpu.SemaphoreType.DMA,
            pltpu.SemaphoreType.DMA,
            pltpu.SemaphoreType.DMA,
            pltpu.SemaphoreType.DMA,
            pltpu.SemaphoreType.DMA,
            pltpu.SemaphoreType.DMA,
            pltpu.VMEM_SHARED((NPAD, 128), jnp.float32),
        ],
    )
    def k(t_hbm, src_hbm, dst_hbm, z_hbm, g_hbm, s_hbm,
          srcf, dstf, dc0, dc1, ts0, ts1, td0, td1, gv, gb,
          sa0, sb0, sa1, sb1, sd0, sd1, s_acc):
        c = lax.axis_index("c")
        s = lax.axis_index("s")
        rbase = s * ROWS_PER_TILE
        ebase = c * E_PER_SC + s * E_PER_TILE
        pltpu.sync_copy(z_hbm.at[pl.ds(rbase, ROWS_PER_TILE)],
                        s_acc.at[pl.ds(rbase, ROWS_PER_TILE)])
        pltpu.sync_copy(z_hbm.at[pl.ds(0, EKG)], gb)
        pltpu.sync_copy(src_hbm.at[pl.ds(ebase, E_PER_TILE)], srcf)
        pltpu.sync_copy(dst_hbm.at[pl.ds(ebase, E_PER_TILE)], dstf)
        plsc.subcore_barrier()

        bufs = ((ts0, td0, dc0, sa0, sb0, sd0), (ts1, td1, dc1, sa1, sb1, sd1))

        def fire(ch, b):
            ts, td, dc, sa, sb, sd = bufs[b]
            h1 = pltpu.async_copy(t_hbm.at[srcf.at[pl.ds(ch * EKG, EKG)]],
                                  ts, sa)
            h2 = pltpu.async_copy(t_hbm.at[dstf.at[pl.ds(ch * EKG, EKG)]],
                                  td, sb)
            h3 = pltpu.async_copy(dst_hbm.at[pl.ds(ebase + ch * EKG, EKG)],
                                  dc, sd)
            return h1, h2, h3

        def drain(ch, b, handles):
            ts, td, dc, _, _, _ = bufs[b]
            for h in handles:
                h.wait()

            def edge(e, _):
                ev = ts[e, pl.ds(0, L)] + td[e, pl.ds(L, L)]
                ev = jnp.where(ev > 0, ev, ev * SLOPE)
                gval = jnp.exp(ev - td[e, pl.ds(2 * L, L)])
                gv[pl.ds(e * L, L)] = gval
                gb[e, pl.ds(0, L)] = gval
                return 0

            lax.fori_loop(0, EKG, edge, 0)
            pltpu.sync_copy(gb, s_acc.at[dc], add=True)
            pltpu.sync_copy(gv,
                            g_hbm.at[pl.ds((ebase + ch * EKG) * L, EKG * L)])

        def pair(i, _):
            c0 = 2 * i
            c1 = 2 * i + 1
            h0 = fire(c0, 0)
            h1 = fire(c1, 1)
            drain(c0, 0, h0)
            drain(c1, 1, h1)
            return 0

        lax.fori_loop(0, N_CHUNK_G // 2, pair, 0)
        plsc.subcore_barrier()
        pltpu.sync_copy(s_acc.at[pl.ds(rbase, ROWS_PER_TILE)],
                        s_hbm.at[c, pl.ds(rbase, ROWS_PER_TILE)])

    return k(tcomb, src, dst, zeros128)


def _make_edge_agg(weighted, nsl):
    scratch = [
        pltpu.VMEM((E_PER_TILE,), jnp.int32),
        pltpu.VMEM((EK,), jnp.int32),
        pltpu.VMEM((EK,), jnp.int32),
        pltpu.VMEM((EK, 128), jnp.float32),
        pltpu.VMEM((EK, 128), jnp.float32),
        pltpu.VMEM((EK * L,), jnp.float32),
        pltpu.VMEM((EK * L,), jnp.float32),
        pltpu.SemaphoreType.DMA,
        pltpu.SemaphoreType.DMA,
        pltpu.SemaphoreType.DMA,
        pltpu.SemaphoreType.DMA,
        pltpu.SemaphoreType.DMA,
        pltpu.SemaphoreType.DMA,
        pltpu.VMEM_SHARED((NPAD, 128), jnp.float32),
    ]

    def body(*args):
        feats = args[:nsl]
        (src_hbm, dst_hbm, g_hbm, z_hbm, out_hbm,
         srcf, dc0, dc1, rows0, rows1, gva, gvb,
         sr0, sr1, sg0, sg1, sd0, sd1, acc) = args[nsl:]
        c = lax.axis_index("c")
        s = lax.axis_index("s")
        rbase = s * ROWS_PER_TILE
        ebase = c * E_PER_SC + s * E_PER_TILE
        pltpu.sync_copy(src_hbm.at[pl.ds(ebase, E_PER_TILE)], srcf)

        bufs = ((rows0, gva, dc0, sr0, sg0, sd0),
                (rows1, gvb, dc1, sr1, sg1, sd1))

        for sl in range(nsl):
            feat_hbm = feats[sl]
            pltpu.sync_copy(z_hbm.at[pl.ds(rbase, ROWS_PER_TILE)],
                            acc.at[pl.ds(rbase, ROWS_PER_TILE)])
            plsc.subcore_barrier()

            def fire(ch, b):
                rows, gv, dc, sr, sg, sd = bufs[b]
                h1 = pltpu.async_copy(
                    feat_hbm.at[srcf.at[pl.ds(ch * EK, EK)]], rows, sr)
                h3 = pltpu.async_copy(dst_hbm.at[pl.ds(ebase + ch * EK, EK)],
                                      dc, sd)
                if weighted:
                    h2 = pltpu.async_copy(
                        g_hbm.at[pl.ds((sl * E + ebase + ch * EK) * L,
                                       EK * L)], gv, sg)
                    return h1, h2, h3
                return h1, h3

            def drain(ch, b, handles):
                rows, gv, dc, _, _, _ = bufs[b]
                for h in handles:
                    h.wait()
                if weighted:
                    def edge(e, _):
                        gv16 = gv[pl.ds(e * L, L)]
                        for j in range(128 // L):
                            rows[e, pl.ds(j * L, L)] = (
                                rows[e, pl.ds(j * L, L)] * gv16)
                        return 0

                    lax.fori_loop(0, EK, edge, 0)
                pltpu.sync_copy(rows, acc.at[dc], add=True)

            def pair(i, _):
                c0 = 2 * i
                c1 = 2 * i + 1
                h0 = fire(c0, 0)
                h1 = fire(c1, 1)
                drain(c0, 0, h0)
                drain(c1, 1, h1)
                return 0

            lax.fori_loop(0, N_CHUNK // 2, pair, 0)
            if N_CHUNK % 2:
                ch = N_CHUNK - 1
                drain(ch, 0, fire(ch, 0))
            plsc.subcore_barrier()
            pltpu.sync_copy(acc.at[pl.ds(rbase, ROWS_PER_TILE)],
                            out_hbm.at[sl, c, pl.ds(rbase, ROWS_PER_TILE)])

    return functools.partial(
        pl.kernel,
        out_type=jax.ShapeDtypeStruct((nsl, NC, NPAD, 128), jnp.float32),
        mesh=_MESH,
        scratch_types=scratch,
    )(body)


_edge_agg_plain = _make_edge_agg(False, 2)
_edge_agg_w5 = _make_edge_agg(True, 5)
_edge_agg_w2 = _make_edge_agg(True, 2)


BM = 1024
NBLK = NPAD // BM


def _gat_proj_body(x_ref, w_ref, alp_ref, arp_ref, feat_ref, el_ref, er_ref,
                   mx_ref):
    feat = x_ref[...] @ w_ref[...]
    feat_ref[...] = feat
    el = feat @ alp_ref[...]
    er = feat @ arp_ref[...]
    el_ref[...] = el
    er_ref[...] = er

    @pl.when(pl.program_id(0) == 0)
    def _():
        mx_ref[...] = jnp.full((8, 128), -1e30, jnp.float32)

    bmax = jnp.max(el, axis=0, keepdims=True)
    mx_ref[...] = jnp.maximum(mx_ref[...], jnp.broadcast_to(bmax, (8, 128)))


def _gat_proj(x, w, alp, arp):
    din, dout = w.shape
    return pl.pallas_call(
        _gat_proj_body,
        grid=(NBLK,),
        in_specs=[
            pl.BlockSpec((BM, din), lambda i: (i, 0)),
            pl.BlockSpec((din, dout), lambda i: (0, 0)),
            pl.BlockSpec((dout, 128), lambda i: (0, 0)),
            pl.BlockSpec((dout, 128), lambda i: (0, 0)),
        ],
        out_specs=[
            pl.BlockSpec((BM, dout), lambda i: (i, 0)),
            pl.BlockSpec((BM, 128), lambda i: (i, 0)),
            pl.BlockSpec((BM, 128), lambda i: (i, 0)),
            pl.BlockSpec((8, 128), lambda i: (0, 0)),
        ],
        out_shape=[
            jax.ShapeDtypeStruct((NPAD, dout), jnp.float32),
            jax.ShapeDtypeStruct((NPAD, 128), jnp.float32),
            jax.ShapeDtypeStruct((NPAD, 128), jnp.float32),
            jax.ShapeDtypeStruct((8, 128), jnp.float32),
        ],
    )(x, w, alp, arp)


def _make_gat_out_body(nhead, dout, act):
    def body(h_ref, r_ref, a0_ref, a1_ref, s0_ref, s1_ref, b_ref, out_ref):
        agg = a0_ref[...] + a1_ref[...]
        sden = s0_ref[...] + s1_ref[...]
        recip = 1.0 / (sden[:, :nhead] + 1e-9)
        if nhead > 1:
            rep = jnp.concatenate(
                [jnp.broadcast_to(recip[:, h:h + 1], (BM, dout))
                 for h in range(nhead)], axis=1)
        else:
            rep = jnp.broadcast_to(recip, (BM, dout))
        out = agg * rep + b_ref[0, :] + h_ref[...] @ r_ref[...]
        if act:
            out = jnp.maximum(out, 0.0)
        out_ref[...] = out
    return body


def _gat_out(h, r, a0, a1, s0, s1, b8, nhead, dout, act):
    din = h.shape[1]
    d = nhead * dout
    return pl.pallas_call(
        _make_gat_out_body(nhead, dout, act),
        grid=(NBLK,),
        in_specs=[
            pl.BlockSpec((BM, din), lambda i: (i, 0)),
            pl.BlockSpec((din, d), lambda i: (0, 0)),
            pl.BlockSpec((BM, d), lambda i: (i, 0)),
            pl.BlockSpec((BM, d), lambda i: (i, 0)),
            pl.BlockSpec((BM, 128), lambda i: (i, 0)),
            pl.BlockSpec((BM, 128), lambda i: (i, 0)),
            pl.BlockSpec((8, d), lambda i: (0, 0)),
        ],
        out_specs=pl.BlockSpec((BM, d), lambda i: (i, 0)),
        out_shape=jax.ShapeDtypeStruct((NPAD, d), jnp.float32),
    )(h, r, a0, a1, s0, s1, b8)


def _gin_mlp_body(h_ref, a0_ref, a1_ref, w1_ref, b1_ref, w2_ref, b2_ref,
                  out_ref):
    h = h_ref[...]
    t = h + a0_ref[...] + a1_ref[...]
    z = jnp.maximum(t @ w1_ref[...] + b1_ref[0, :], 0.0)
    out_ref[...] = z @ w2_ref[...] + b2_ref[0, :] + h


def _gin_mlp(h, a0, a1, w1, b18, w2, b28):
    return pl.pallas_call(
        _gin_mlp_body,
        grid=(NBLK,),
        in_specs=[
            pl.BlockSpec((BM, SUB), lambda i: (i, 0)),
            pl.BlockSpec((BM, SUB), lambda i: (i, 0)),
            pl.BlockSpec((BM, SUB), lambda i: (i, 0)),
            pl.BlockSpec((SUB, SUB), lambda i: (0, 0)),
            pl.BlockSpec((8, SUB), lambda i: (0, 0)),
            pl.BlockSpec((SUB, SUB), lambda i: (0, 0)),
            pl.BlockSpec((8, SUB), lambda i: (0, 0)),
        ],
        out_specs=pl.BlockSpec((BM, SUB), lambda i: (i, 0)),
        out_shape=jax.ShapeDtypeStruct((NPAD, SUB), jnp.float32),
    )(h, a0, a1, w1, b18, w2, b28)


def _pool_body(p_ref, x_ref, out_ref):
    @pl.when(pl.program_id(0) == 0)
    def _():
        out_ref[...] = jnp.zeros_like(out_ref)

    out_ref[...] += lax.dot_general(
        p_ref[...], x_ref[...], (((0,), (0,)), ((), ())),
        preferred_element_type=jnp.float32)


def _pool(p128, x):
    dx = x.shape[1]
    return pl.pallas_call(
        _pool_body,
        grid=(NBLK,),
        in_specs=[
            pl.BlockSpec((BM, 128), lambda i: (i, 0)),
            pl.BlockSpec((BM, dx), lambda i: (i, 0)),
        ],
        out_specs=pl.BlockSpec((128, dx), lambda i: (0, 0)),
        out_shape=jax.ShapeDtypeStruct((128, dx), jnp.float32),
    )(p128, x)


def _head_body(ps_ref, pf_ref, hm_ref, fpc_ref, adm_ref, admb_ref, adf_ref,
               adfb_ref, p1_ref, p2_ref, p3_ref, out_ref):
    pf = pf_ref[...]
    cnt = pf[:, SUB:SUB + 128]
    cnt2 = jnp.concatenate([cnt, cnt], axis=1)
    xf = pf[:, :SUB] / jnp.maximum(cnt2, 1.0)
    y = jnp.concatenate([ps_ref[...], xf], axis=1)
    y = y + hm_ref[...] @ adm_ref[...] + admb_ref[0, :]
    y = y + fpc_ref[...] @ adf_ref[...] + adfb_ref[0, :]
    z = jnp.maximum(y @ p1_ref[...], 0.0)
    z = jnp.maximum(z @ p2_ref[...], 0.0)
    out_ref[...] = z @ p3_ref[...]


def _head(ps, pf, hm, fpc, adm, admb8, adf, adfb8, p1, p2, p3p):
    return pl.pallas_call(
        _head_body,
        out_shape=jax.ShapeDtypeStruct((B, 128), jnp.float32),
    )(ps, pf, hm, fpc, adm, admb8, adf, adfb8, p1, p2, p3p)


def _row8(v):
    return jnp.broadcast_to(v[None, :], (8, v.shape[0]))


def _blockdiag_attn(a, dout):
    nh = a.shape[0]
    rows = jnp.arange(nh * dout)
    return jnp.where(jnp.arange(128)[None, :] == (rows // dout)[:, None],
                     a.reshape(-1)[:, None], 0.0).astype(jnp.float32)


def _gat_layer(h, src4, dst4, W, al, ar, b, R, nhead, dout, act, zeros128):
    d = nhead * dout
    alp = _blockdiag_attn(al, dout)
    arp = _blockdiag_attn(ar, dout)
    feat, el128, er128, mx = _gat_proj(h, W, alp, arp)
    lane_pad = jnp.where(jnp.arange(L) < nhead, 0.0, -1e30).astype(jnp.float32)
    el16 = el128[:, :L] + lane_pad[None, :]
    cpre = mx[0:1, :L] + er128[:, :L]
    c16 = jnp.where(cpre > 0, cpre, cpre * SLOPE)
    er16 = er128[:, :L]
    tcomb = jnp.concatenate(
        [el16, er16, c16, jnp.zeros((NPAD, 128 - 3 * L), jnp.float32)], axis=1)

    gf, s_part = _sc_edge_gate(tcomb, src4, dst4, zeros128)
    g = gf.reshape(E, L)
    s0 = s_part[0]
    s1 = s_part[1]

    nslice = d // 128
    featperm = feat.reshape(NPAD, nslice, 128).transpose(1, 0, 2)
    lanes = list(range(nhead)) if nhead > 1 else [0] * nslice
    gxs = jnp.concatenate(
        [jnp.broadcast_to(g[:, ln:ln + 1], (E, L)).reshape(E * L)
         for ln in lanes])
    agg_fn = _edge_agg_w5 if nslice == 5 else _edge_agg_w2
    parts = agg_fn(*[featperm[p] for p in range(nslice)],
                   src4, dst4, gxs, zeros128)
    a0 = jnp.concatenate([parts[p, 0] for p in range(nslice)], axis=1)
    a1 = jnp.concatenate([parts[p, 1] for p in range(nslice)], axis=1)

    return _gat_out(h, R, a0, a1, s0, s1, _row8(b), nhead, dout, act)


def _gin_layer(h, src4, dst4, w1, b1, w2, b2, zeros128, g_dummy):
    hperm = h.reshape(NPAD, 2, 128).transpose(1, 0, 2)
    parts = _edge_agg_plain(hperm[0], hperm[1], src4, dst4, g_dummy, zeros128)
    a0 = jnp.concatenate([parts[0, 0], parts[1, 0]], axis=1)
    a1 = jnp.concatenate([parts[0, 1], parts[1, 1]], axis=1)
    return _gin_mlp(h, a0, a1, w1, _row8(b1), w2, _row8(b2))


def kernel(node_feature, h_MolCLR, maccs, morgan, params, edge_index,
           node_subgraph, graph_ids):
    p = params
    src4 = edge_index[0].astype(jnp.int32)
    dst4 = edge_index[1].astype(jnp.int32)

    zeros128 = jnp.zeros((NPAD, 128), jnp.float32)
    g_dummy = jnp.zeros((E * L,), jnp.float32)

    idx_pad = jnp.zeros((NPAD,), jnp.int32).at[:N].set(
        node_subgraph.astype(jnp.int32))
    x_sub = _sc_gather(p['embed'], idx_pad)

    x_sub = _gin_layer(x_sub, src4, dst4, p['gin_w1_0'], p['gin_b1_0'],
                       p['gin_w2_0'], p['gin_b2_0'], zeros128, g_dummy)
    x_sub = _gin_layer(x_sub, src4, dst4, p['gin_w1_1'], p['gin_b1_1'],
                       p['gin_w2_1'], p['gin_b2_1'], zeros128, g_dummy)

    gid_pad = jnp.full((NPAD,), -1, jnp.int32).at[:N].set(
        graph_ids.astype(jnp.int32))
    p128 = (gid_pad[:, None] == jnp.arange(128)[None, :]).astype(jnp.float32)

    pool_sub = _pool(p128, x_sub)[:B, :]

    hf = jnp.zeros((NPAD, FEAT), jnp.float32).at[:N].set(node_feature)
    x_feat = _gat_layer(hf, src4, dst4, p['gat_W_0'],
                        p['gat_al_0'], p['gat_ar_0'], p['gat_b_0'],
                        p['gat_R_0'], NH1, H1, True, zeros128)
    x_feat = _gat_layer(x_feat, src4, dst4, p['gat_W_1'],
                        p['gat_al_1'], p['gat_ar_1'], p['gat_b_1'],
                        p['gat_R_1'], NH2, O2, False, zeros128)

    xc = jnp.concatenate([x_feat, jnp.where(gid_pad >= 0, 1.0, 0.0)[:, None]
                          * jnp.ones((NPAD, 128), jnp.float32)], axis=1)
    pool_feat = _pool(p128, xc)[:B, :]

    fpc = jnp.zeros((B, 384), jnp.float32)
    fpc = fpc.at[:, :167].set(maccs.astype(jnp.float32))
    fpc = fpc.at[:, 167:367].set(morgan.astype(jnp.float32))
    adf = jnp.zeros((384, MOL), jnp.float32).at[:367].set(p['adF_w'])
    p3p = jnp.zeros((128, 128), jnp.float32).at[:, :11].set(p['p3'])

    out = _head(pool_sub, pool_feat, h_MolCLR, fpc,
                p['adM_w'], _row8(p['adM_b']), adf, _row8(p['adF_b']),
                p['p1'], p['p2'], p3p)
    return out[:, :11]


# speedup vs baseline: 1.2048x; 1.2048x over previous
import functools
import jax
import jax.numpy as jnp
from jax import lax
from jax.experimental import pallas as pl
from jax.experimental.pallas import tpu as pltpu
from jax.experimental.pallas import tpu_sc as plsc

N = 10000; E = 320000; B = 64; FEAT = 128; SUB = 256; MOL = 512; NF = 100000
NH1 = 5; H1 = 128; NH2 = 1; O2 = 256; SLOPE = 0.1

NC = 2
NS = 16
NW = NC * NS
L = 16
NPAD = 10240
EK = 80
E_PER_SC = E // NC
E_PER_TILE = E // NW
N_CHUNK = E_PER_TILE // EK
ROWS_PER_TILE = NPAD // NS

_MESH = plsc.VectorSubcoreMesh(core_axis_name="c", subcore_axis_name="s")


def _sc_gather(table, idx):
    V, D = table.shape
    n = idx.shape[0]
    per_w = n // NW
    n_chunk = per_w // EK

    @functools.partial(
        pl.kernel,
        out_type=jax.ShapeDtypeStruct((n, D), jnp.float32),
        mesh=_MESH,
        scratch_types=[
            pltpu.VMEM((EK,), jnp.int32),
            pltpu.VMEM((EK, D), jnp.float32),
            pltpu.SemaphoreType.DMA,
        ],
    )
    def k(table_hbm, idx_hbm, out_hbm, idx_v, rows_v, sem):
        wid = lax.axis_index("s") * NC + lax.axis_index("c")
        base = wid * per_w

        def chunk(i, _):
            off = base + i * EK
            pltpu.sync_copy(idx_hbm.at[pl.ds(off, EK)], idx_v)
            pltpu.async_copy(table_hbm.at[idx_v], rows_v, sem).wait()
            pltpu.sync_copy(rows_v, out_hbm.at[pl.ds(off, EK)])
            return 0

        lax.fori_loop(0, n_chunk, chunk, 0)

    return k(table, idx)


EKG = 40
N_CHUNK_G = E_PER_TILE // EKG


def _sc_edge_gate(tcomb, src, dst, zeros128):
    @functools.partial(
        pl.kernel,
        out_type=(
            jax.ShapeDtypeStruct((E * L,), jnp.float32),
            jax.ShapeDtypeStruct((NC, NPAD, 128), jnp.float32),
        ),
        mesh=_MESH,
        scratch_types=[
            pltpu.VMEM((E_PER_TILE,), jnp.int32),
            pltpu.VMEM((E_PER_TILE,), jnp.int32),
            pltpu.VMEM((EKG,), jnp.int32),
            pltpu.VMEM((EKG,), jnp.int32),
            pltpu.VMEM((EKG, 128), jnp.float32),
            pltpu.VMEM((EKG, 128), jnp.float32),
            pltpu.VMEM((EKG, 128), jnp.float32),
            pltpu.VMEM((EKG, 128), jnp.float32),
            pltpu.VMEM((EKG * L,), jnp.float32),
            pltpu.VMEM((EKG, 128), jnp.float32),
            pltpu.SemaphoreType.DMA,
            pltpu.SemaphoreType.DMA,
            pltpu.SemaphoreType.DMA,
            pltpu.SemaphoreType.DMA,
            pltpu.SemaphoreType.DMA,
            pltpu.SemaphoreType.DMA,
            pltpu.VMEM_SHARED((NPAD, 128), jnp.float32),
        ],
    )
    def k(t_hbm, src_hbm, dst_hbm, z_hbm, g_hbm, s_hbm,
          srcf, dstf, dc0, dc1, ts0, ts1, td0, td1, gv, gb,
          sa0, sb0, sa1, sb1, sd0, sd1, s_acc):
        c = lax.axis_index("c")
        s = lax.axis_index("s")
        rbase = s * ROWS_PER_TILE
        ebase = c * E_PER_SC + s * E_PER_TILE
        pltpu.sync_copy(z_hbm.at[pl.ds(rbase, ROWS_PER_TILE)],
                        s_acc.at[pl.ds(rbase, ROWS_PER_TILE)])
        pltpu.sync_copy(z_hbm.at[pl.ds(0, EKG)], gb)
        pltpu.sync_copy(src_hbm.at[pl.ds(ebase, E_PER_TILE)], srcf)
        pltpu.sync_copy(dst_hbm.at[pl.ds(ebase, E_PER_TILE)], dstf)
        plsc.subcore_barrier()

        bufs = ((ts0, td0, dc0, sa0, sb0, sd0), (ts1, td1, dc1, sa1, sb1, sd1))

        def fire(ch, b):
            ts, td, dc, sa, sb, sd = bufs[b]
            h1 = pltpu.async_copy(t_hbm.at[srcf.at[pl.ds(ch * EKG, EKG)]],
                                  ts, sa)
            h2 = pltpu.async_copy(t_hbm.at[dstf.at[pl.ds(ch * EKG, EKG)]],
                                  td, sb)
            h3 = pltpu.async_copy(dst_hbm.at[pl.ds(ebase + ch * EKG, EKG)],
                                  dc, sd)
            return h1, h2, h3

        def drain(ch, b, handles):
            ts, td, dc, _, _, _ = bufs[b]
            for h in handles:
                h.wait()

            def edge(e, _):
                ev = ts[e, pl.ds(0, L)] + td[e, pl.ds(L, L)]
                ev = jnp.where(ev > 0, ev, ev * SLOPE)
                gval = jnp.exp(ev - td[e, pl.ds(2 * L, L)])
                gv[pl.ds(e * L, L)] = gval
                gb[e, pl.ds(0, L)] = gval
                return 0

            lax.fori_loop(0, EKG, edge, 0)
            pltpu.sync_copy(gb, s_acc.at[dc], add=True)
            pltpu.sync_copy(gv,
                            g_hbm.at[pl.ds((ebase + ch * EKG) * L, EKG * L)])

        def pair(i, _):
            c0 = 2 * i
            c1 = 2 * i + 1
            h0 = fire(c0, 0)
            h1 = fire(c1, 1)
            drain(c0, 0, h0)
            drain(c1, 1, h1)
            return 0

        lax.fori_loop(0, N_CHUNK_G // 2, pair, 0)
        plsc.subcore_barrier()
        pltpu.sync_copy(s_acc.at[pl.ds(rbase, ROWS_PER_TILE)],
                        s_hbm.at[c, pl.ds(rbase, ROWS_PER_TILE)])

    return k(tcomb, src, dst, zeros128)


def _make_edge_agg(weighted):
    scratch = [
        pltpu.VMEM((E_PER_TILE,), jnp.int32),
        pltpu.VMEM((EK,), jnp.int32),
        pltpu.VMEM((EK,), jnp.int32),
        pltpu.VMEM((EK, 128), jnp.float32),
        pltpu.VMEM((EK, 128), jnp.float32),
        pltpu.VMEM((EK * L,), jnp.float32),
        pltpu.VMEM((EK * L,), jnp.float32),
        pltpu.SemaphoreType.DMA,
        pltpu.SemaphoreType.DMA,
        pltpu.SemaphoreType.DMA,
        pltpu.SemaphoreType.DMA,
        pltpu.SemaphoreType.DMA,
        pltpu.SemaphoreType.DMA,
        pltpu.VMEM_SHARED((NPAD, 128), jnp.float32),
    ]

    def body(feat_hbm, src_hbm, dst_hbm, g_hbm, z_hbm, out_hbm,
             srcf, dc0, dc1, rows0, rows1, gva, gvb,
             sr0, sr1, sg0, sg1, sd0, sd1, acc):
        c = lax.axis_index("c")
        s = lax.axis_index("s")
        rbase = s * ROWS_PER_TILE
        ebase = c * E_PER_SC + s * E_PER_TILE
        pltpu.sync_copy(z_hbm.at[pl.ds(rbase, ROWS_PER_TILE)],
                        acc.at[pl.ds(rbase, ROWS_PER_TILE)])
        pltpu.sync_copy(src_hbm.at[pl.ds(ebase, E_PER_TILE)], srcf)
        plsc.subcore_barrier()

        bufs = ((rows0, gva, dc0, sr0, sg0, sd0),
                (rows1, gvb, dc1, sr1, sg1, sd1))

        def fire(ch, b):
            rows, gv, dc, sr, sg, sd = bufs[b]
            h1 = pltpu.async_copy(
                feat_hbm.at[srcf.at[pl.ds(ch * EK, EK)]], rows, sr)
            h3 = pltpu.async_copy(dst_hbm.at[pl.ds(ebase + ch * EK, EK)],
                                  dc, sd)
            if weighted:
                h2 = pltpu.async_copy(
                    g_hbm.at[pl.ds((ebase + ch * EK) * L, EK * L)], gv, sg)
                return h1, h2, h3
            return h1, h3

        def drain(ch, b, handles):
            rows, gv, dc, _, _, _ = bufs[b]
            for h in handles:
                h.wait()
            if weighted:
                def edge(e, _):
                    gv16 = gv[pl.ds(e * L, L)]
                    for j in range(128 // L):
                        rows[e, pl.ds(j * L, L)] = rows[e, pl.ds(j * L, L)] * gv16
                    return 0

                lax.fori_loop(0, EK, edge, 0)
            pltpu.sync_copy(rows, acc.at[dc], add=True)

        def pair(i, _):
            c0 = 2 * i
            c1 = 2 * i + 1
            h0 = fire(c0, 0)
            h1 = fire(c1, 1)
            drain(c0, 0, h0)
            drain(c1, 1, h1)
            return 0

        lax.fori_loop(0, N_CHUNK // 2, pair, 0)
        if N_CHUNK % 2:
            ch = N_CHUNK - 1
            drain(ch, 0, fire(ch, 0))
        plsc.subcore_barrier()
        pltpu.sync_copy(acc.at[pl.ds(rbase, ROWS_PER_TILE)],
                        out_hbm.at[c, pl.ds(rbase, ROWS_PER_TILE)])

    return functools.partial(
        pl.kernel,
        out_type=jax.ShapeDtypeStruct((NC, NPAD, 128), jnp.float32),
        mesh=_MESH,
        scratch_types=scratch,
    )(body)


_edge_agg_plain = _make_edge_agg(False)
_edge_agg_w = _make_edge_agg(True)


BM = 1024
NBLK = NPAD // BM


def _gat_proj_body(x_ref, w_ref, alp_ref, arp_ref, feat_ref, el_ref, er_ref,
                   mx_ref):
    feat = x_ref[...] @ w_ref[...]
    feat_ref[...] = feat
    el = feat @ alp_ref[...]
    er = feat @ arp_ref[...]
    el_ref[...] = el
    er_ref[...] = er

    @pl.when(pl.program_id(0) == 0)
    def _():
        mx_ref[...] = jnp.full((8, 128), -1e30, jnp.float32)

    bmax = jnp.max(el, axis=0, keepdims=True)
    mx_ref[...] = jnp.maximum(mx_ref[...], jnp.broadcast_to(bmax, (8, 128)))


def _gat_proj(x, w, alp, arp):
    din, dout = w.shape
    return pl.pallas_call(
        _gat_proj_body,
        grid=(NBLK,),
        in_specs=[
            pl.BlockSpec((BM, din), lambda i: (i, 0)),
            pl.BlockSpec((din, dout), lambda i: (0, 0)),
            pl.BlockSpec((dout, 128), lambda i: (0, 0)),
            pl.BlockSpec((dout, 128), lambda i: (0, 0)),
        ],
        out_specs=[
            pl.BlockSpec((BM, dout), lambda i: (i, 0)),
            pl.BlockSpec((BM, 128), lambda i: (i, 0)),
            pl.BlockSpec((BM, 128), lambda i: (i, 0)),
            pl.BlockSpec((8, 128), lambda i: (0, 0)),
        ],
        out_shape=[
            jax.ShapeDtypeStruct((NPAD, dout), jnp.float32),
            jax.ShapeDtypeStruct((NPAD, 128), jnp.float32),
            jax.ShapeDtypeStruct((NPAD, 128), jnp.float32),
            jax.ShapeDtypeStruct((8, 128), jnp.float32),
        ],
    )(x, w, alp, arp)


def _make_gat_out_body(nhead, dout, act):
    def body(h_ref, r_ref, a0_ref, a1_ref, s0_ref, s1_ref, b_ref, out_ref):
        agg = a0_ref[...] + a1_ref[...]
        sden = s0_ref[...] + s1_ref[...]
        recip = 1.0 / (sden[:, :nhead] + 1e-9)
        if nhead > 1:
            rep = jnp.concatenate(
                [jnp.broadcast_to(recip[:, h:h + 1], (BM, dout))
                 for h in range(nhead)], axis=1)
        else:
            rep = jnp.broadcast_to(recip, (BM, dout))
        out = agg * rep + b_ref[0, :] + h_ref[...] @ r_ref[...]
        if act:
            out = jnp.maximum(out, 0.0)
        out_ref[...] = out
    return body


def _gat_out(h, r, a0, a1, s0, s1, b8, nhead, dout, act):
    din = h.shape[1]
    d = nhead * dout
    return pl.pallas_call(
        _make_gat_out_body(nhead, dout, act),
        grid=(NBLK,),
        in_specs=[
            pl.BlockSpec((BM, din), lambda i: (i, 0)),
            pl.BlockSpec((din, d), lambda i: (0, 0)),
            pl.BlockSpec((BM, d), lambda i: (i, 0)),
            pl.BlockSpec((BM, d), lambda i: (i, 0)),
            pl.BlockSpec((BM, 128), lambda i: (i, 0)),
            pl.BlockSpec((BM, 128), lambda i: (i, 0)),
            pl.BlockSpec((8, d), lambda i: (0, 0)),
        ],
        out_specs=pl.BlockSpec((BM, d), lambda i: (i, 0)),
        out_shape=jax.ShapeDtypeStruct((NPAD, d), jnp.float32),
    )(h, r, a0, a1, s0, s1, b8)


def _gin_mlp_body(h_ref, a0_ref, a1_ref, w1_ref, b1_ref, w2_ref, b2_ref,
                  out_ref):
    h = h_ref[...]
    t = h + a0_ref[...] + a1_ref[...]
    z = jnp.maximum(t @ w1_ref[...] + b1_ref[0, :], 0.0)
    out_ref[...] = z @ w2_ref[...] + b2_ref[0, :] + h


def _gin_mlp(h, a0, a1, w1, b18, w2, b28):
    return pl.pallas_call(
        _gin_mlp_body,
        grid=(NBLK,),
        in_specs=[
            pl.BlockSpec((BM, SUB), lambda i: (i, 0)),
            pl.BlockSpec((BM, SUB), lambda i: (i, 0)),
            pl.BlockSpec((BM, SUB), lambda i: (i, 0)),
            pl.BlockSpec((SUB, SUB), lambda i: (0, 0)),
            pl.BlockSpec((8, SUB), lambda i: (0, 0)),
            pl.BlockSpec((SUB, SUB), lambda i: (0, 0)),
            pl.BlockSpec((8, SUB), lambda i: (0, 0)),
        ],
        out_specs=pl.BlockSpec((BM, SUB), lambda i: (i, 0)),
        out_shape=jax.ShapeDtypeStruct((NPAD, SUB), jnp.float32),
    )(h, a0, a1, w1, b18, w2, b28)


def _pool_body(p_ref, x_ref, out_ref):
    @pl.when(pl.program_id(0) == 0)
    def _():
        out_ref[...] = jnp.zeros_like(out_ref)

    out_ref[...] += lax.dot_general(
        p_ref[...], x_ref[...], (((0,), (0,)), ((), ())),
        preferred_element_type=jnp.float32)


def _pool(p128, x):
    dx = x.shape[1]
    return pl.pallas_call(
        _pool_body,
        grid=(NBLK,),
        in_specs=[
            pl.BlockSpec((BM, 128), lambda i: (i, 0)),
            pl.BlockSpec((BM, dx), lambda i: (i, 0)),
        ],
        out_specs=pl.BlockSpec((128, dx), lambda i: (0, 0)),
        out_shape=jax.ShapeDtypeStruct((128, dx), jnp.float32),
    )(p128, x)


def _head_body(ps_ref, pf_ref, hm_ref, fpc_ref, adm_ref, admb_ref, adf_ref,
               adfb_ref, p1_ref, p2_ref, p3_ref, out_ref):
    pf = pf_ref[...]
    cnt = pf[:, SUB:SUB + 128]
    cnt2 = jnp.concatenate([cnt, cnt], axis=1)
    xf = pf[:, :SUB] / jnp.maximum(cnt2, 1.0)
    y = jnp.concatenate([ps_ref[...], xf], axis=1)
    y = y + hm_ref[...] @ adm_ref[...] + admb_ref[0, :]
    y = y + fpc_ref[...] @ adf_ref[...] + adfb_ref[0, :]
    z = jnp.maximum(y @ p1_ref[...], 0.0)
    z = jnp.maximum(z @ p2_ref[...], 0.0)
    out_ref[...] = z @ p3_ref[...]


def _head(ps, pf, hm, fpc, adm, admb8, adf, adfb8, p1, p2, p3p):
    return pl.pallas_call(
        _head_body,
        out_shape=jax.ShapeDtypeStruct((B, 128), jnp.float32),
    )(ps, pf, hm, fpc, adm, admb8, adf, adfb8, p1, p2, p3p)


def _row8(v):
    return jnp.broadcast_to(v[None, :], (8, v.shape[0]))


def _blockdiag_attn(a, dout):
    nh = a.shape[0]
    rows = jnp.arange(nh * dout)
    return jnp.where(jnp.arange(128)[None, :] == (rows // dout)[:, None],
                     a.reshape(-1)[:, None], 0.0).astype(jnp.float32)


def _gat_layer(h, src4, dst4, W, al, ar, b, R, nhead, dout, act, zeros128):
    d = nhead * dout
    alp = _blockdiag_attn(al, dout)
    arp = _blockdiag_attn(ar, dout)
    feat, el128, er128, mx = _gat_proj(h, W, alp, arp)
    lane_pad = jnp.where(jnp.arange(L) < nhead, 0.0, -1e30).astype(jnp.float32)
    el16 = el128[:, :L] + lane_pad[None, :]
    cpre = mx[0:1, :L] + er128[:, :L]
    c16 = jnp.where(cpre > 0, cpre, cpre * SLOPE)
    er16 = er128[:, :L]
    tcomb = jnp.concatenate(
        [el16, er16, c16, jnp.zeros((NPAD, 128 - 3 * L), jnp.float32)], axis=1)

    gf, s_part = _sc_edge_gate(tcomb, src4, dst4, zeros128)
    g = gf.reshape(E, L)
    s0 = s_part[0]
    s1 = s_part[1]

    nslice = d // 128
    featperm = feat.reshape(NPAD, nslice, 128).transpose(1, 0, 2)
    parts0 = []
    parts1 = []
    for p in range(nslice):
        lane = p if nhead > 1 else 0
        gx = jnp.broadcast_to(g[:, lane:lane + 1], (E, L)).reshape(E * L)
        part = _edge_agg_w(featperm[p], src4, dst4, gx, zeros128)
        parts0.append(part[0])
        parts1.append(part[1])
    a0 = jnp.concatenate(parts0, axis=1)
    a1 = jnp.concatenate(parts1, axis=1)

    return _gat_out(h, R, a0, a1, s0, s1, _row8(b), nhead, dout, act)


def _gin_layer(h, src4, dst4, w1, b1, w2, b2, zeros128, g_dummy):
    hperm = h.reshape(NPAD, 2, 128).transpose(1, 0, 2)
    pa = _edge_agg_plain(hperm[0], src4, dst4, g_dummy, zeros128)
    pb = _edge_agg_plain(hperm[1], src4, dst4, g_dummy, zeros128)
    a0 = jnp.concatenate([pa[0], pb[0]], axis=1)
    a1 = jnp.concatenate([pa[1], pb[1]], axis=1)
    return _gin_mlp(h, a0, a1, w1, _row8(b1), w2, _row8(b2))


def kernel(node_feature, h_MolCLR, maccs, morgan, params, edge_index,
           node_subgraph, graph_ids):
    p = params
    src4 = edge_index[0].astype(jnp.int32)
    dst4 = edge_index[1].astype(jnp.int32)

    zeros128 = jnp.zeros((NPAD, 128), jnp.float32)
    g_dummy = jnp.zeros((E * L,), jnp.float32)

    idx_pad = jnp.zeros((NPAD,), jnp.int32).at[:N].set(
        node_subgraph.astype(jnp.int32))
    x_sub = _sc_gather(p['embed'], idx_pad)

    x_sub = _gin_layer(x_sub, src4, dst4, p['gin_w1_0'], p['gin_b1_0'],
                       p['gin_w2_0'], p['gin_b2_0'], zeros128, g_dummy)
    x_sub = _gin_layer(x_sub, src4, dst4, p['gin_w1_1'], p['gin_b1_1'],
                       p['gin_w2_1'], p['gin_b2_1'], zeros128, g_dummy)

    gid_pad = jnp.full((NPAD,), -1, jnp.int32).at[:N].set(
        graph_ids.astype(jnp.int32))
    p128 = (gid_pad[:, None] == jnp.arange(128)[None, :]).astype(jnp.float32)

    pool_sub = _pool(p128, x_sub)[:B, :]

    hf = jnp.zeros((NPAD, FEAT), jnp.float32).at[:N].set(node_feature)
    x_feat = _gat_layer(hf, src4, dst4, p['gat_W_0'],
                        p['gat_al_0'], p['gat_ar_0'], p['gat_b_0'],
                        p['gat_R_0'], NH1, H1, True, zeros128)
    x_feat = _gat_layer(x_feat, src4, dst4, p['gat_W_1'],
                        p['gat_al_1'], p['gat_ar_1'], p['gat_b_1'],
                        p['gat_R_1'], NH2, O2, False, zeros128)

    xc = jnp.concatenate([x_feat, jnp.where(gid_pad >= 0, 1.0, 0.0)[:, None]
                          * jnp.ones((NPAD, 128), jnp.float32)], axis=1)
    pool_feat = _pool(p128, xc)[:B, :]

    fpc = jnp.zeros((B, 384), jnp.float32)
    fpc = fpc.at[:, :167].set(maccs.astype(jnp.float32))
    fpc = fpc.at[:, 167:367].set(morgan.astype(jnp.float32))
    adf = jnp.zeros((384, MOL), jnp.float32).at[:367].set(p['adF_w'])
    p3p = jnp.zeros((128, 128), jnp.float32).at[:, :11].set(p['p3'])

    out = _head(pool_sub, pool_feat, h_MolCLR, fpc,
                p['adM_w'], _row8(p['adM_b']), adf, _row8(p['adF_b']),
                p['p1'], p['p2'], p3p)
    return out[:, :11]


# speedup vs baseline: 1.2363x; 1.0262x over previous
import functools
import jax
import jax.numpy as jnp
from jax import lax
from jax.experimental import pallas as pl
from jax.experimental.pallas import tpu as pltpu
from jax.experimental.pallas import tpu_sc as plsc

N = 10000; E = 320000; B = 64; FEAT = 128; SUB = 256; MOL = 512; NF = 100000
NH1 = 5; H1 = 128; NH2 = 1; O2 = 256; SLOPE = 0.1

NC = 2
NS = 16
NW = NC * NS
L = 16
NPAD = 10240
EK = 80
E_PER_SC = E // NC
E_PER_TILE = E // NW
N_CHUNK = E_PER_TILE // EK
ROWS_PER_TILE = NPAD // NS

_MESH = plsc.VectorSubcoreMesh(core_axis_name="c", subcore_axis_name="s")


def _sc_gather(table, idx):
    V, D = table.shape
    n = idx.shape[0]
    per_w = n // NW
    n_chunk = per_w // EK

    @functools.partial(
        pl.kernel,
        out_type=jax.ShapeDtypeStruct((n, D), jnp.float32),
        mesh=_MESH,
        scratch_types=[
            pltpu.VMEM((EK,), jnp.int32),
            pltpu.VMEM((EK, D), jnp.float32),
            pltpu.SemaphoreType.DMA,
        ],
    )
    def k(table_hbm, idx_hbm, out_hbm, idx_v, rows_v, sem):
        wid = lax.axis_index("s") * NC + lax.axis_index("c")
        base = wid * per_w

        def chunk(i, _):
            off = base + i * EK
            pltpu.sync_copy(idx_hbm.at[pl.ds(off, EK)], idx_v)
            pltpu.async_copy(table_hbm.at[idx_v], rows_v, sem).wait()
            pltpu.sync_copy(rows_v, out_hbm.at[pl.ds(off, EK)])
            return 0

        lax.fori_loop(0, n_chunk, chunk, 0)

    return k(table, idx)


EKG = 40
N_CHUNK_G = E_PER_TILE // EKG


def _sc_edge_gate(tcomb, src, dst, zeros128):
    @functools.partial(
        pl.kernel,
        out_type=(
            jax.ShapeDtypeStruct((E * L,), jnp.float32),
            jax.ShapeDtypeStruct((NC, NPAD, 128), jnp.float32),
        ),
        mesh=_MESH,
        scratch_types=[
            pltpu.VMEM((E_PER_TILE,), jnp.int32),
            pltpu.VMEM((E_PER_TILE,), jnp.int32),
            pltpu.VMEM((EKG,), jnp.int32),
            pltpu.VMEM((EKG,), jnp.int32),
            pltpu.VMEM((EKG, 128), jnp.float32),
            pltpu.VMEM((EKG, 128), jnp.float32),
            pltpu.VMEM((EKG, 128), jnp.float32),
            pltpu.VMEM((EKG, 128), jnp.float32),
            pltpu.VMEM((EKG * L,), jnp.float32),
            pltpu.VMEM((EKG, 128), jnp.float32),
            pltpu.SemaphoreType.DMA,
            pltpu.SemaphoreType.DMA,
            pltpu.SemaphoreType.DMA,
            pltpu.SemaphoreType.DMA,
            pltpu.SemaphoreType.DMA,
            pltpu.SemaphoreType.DMA,
            pltpu.VMEM_SHARED((NPAD, 128), jnp.float32),
        ],
    )
    def k(t_hbm, src_hbm, dst_hbm, z_hbm, g_hbm, s_hbm,
          srcf, dstf, dc0, dc1, ts0, ts1, td0, td1, gv, gb,
          sa0, sb0, sa1, sb1, sd0, sd1, s_acc):
        c = lax.axis_index("c")
        s = lax.axis_index("s")
        rbase = s * ROWS_PER_TILE
        ebase = c * E_PER_SC + s * E_PER_TILE
        pltpu.sync_copy(z_hbm.at[pl.ds(rbase, ROWS_PER_TILE)],
                        s_acc.at[pl.ds(rbase, ROWS_PER_TILE)])
        pltpu.sync_copy(z_hbm.at[pl.ds(0, EKG)], gb)
        pltpu.sync_copy(src_hbm.at[pl.ds(ebase, E_PER_TILE)], srcf)
        pltpu.sync_copy(dst_hbm.at[pl.ds(ebase, E_PER_TILE)], dstf)
        plsc.subcore_barrier()

        bufs = ((ts0, td0, dc0, sa0, sb0, sd0), (ts1, td1, dc1, sa1, sb1, sd1))

        def fire(ch, b):
            ts, td, dc, sa, sb, sd = bufs[b]
            h1 = pltpu.async_copy(t_hbm.at[srcf.at[pl.ds(ch * EKG, EKG)]],
                                  ts, sa)
            h2 = pltpu.async_copy(t_hbm.at[dstf.at[pl.ds(ch * EKG, EKG)]],
                                  td, sb)
            h3 = pltpu.async_copy(dst_hbm.at[pl.ds(ebase + ch * EKG, EKG)],
                                  dc, sd)
            return h1, h2, h3

        def drain(ch, b, handles):
            ts, td, dc, _, _, _ = bufs[b]
            for h in handles:
                h.wait()

            def edge(e, _):
                ev = ts[e, pl.ds(0, L)] + td[e, pl.ds(L, L)]
                ev = jnp.where(ev > 0, ev, ev * SLOPE)
                gval = jnp.exp(ev - td[e, pl.ds(2 * L, L)])
                gv[pl.ds(e * L, L)] = gval
                gb[e, pl.ds(0, L)] = gval
                return 0

            lax.fori_loop(0, EKG, edge, 0)
            pltpu.sync_copy(gb, s_acc.at[dc], add=True)
            pltpu.sync_copy(gv,
                            g_hbm.at[pl.ds((ebase + ch * EKG) * L, EKG * L)])

        def pair(i, _):
            c0 = 2 * i
            c1 = 2 * i + 1
            h0 = fire(c0, 0)
            h1 = fire(c1, 1)
            drain(c0, 0, h0)
            drain(c1, 1, h1)
            return 0

        lax.fori_loop(0, N_CHUNK_G // 2, pair, 0)
        plsc.subcore_barrier()
        pltpu.sync_copy(s_acc.at[pl.ds(rbase, ROWS_PER_TILE)],
                        s_hbm.at[c, pl.ds(rbase, ROWS_PER_TILE)])

    return k(tcomb, src, dst, zeros128)


def _make_edge_agg(weighted):
    scratch = [
        pltpu.VMEM((E_PER_TILE,), jnp.int32),
        pltpu.VMEM((EK,), jnp.int32),
        pltpu.VMEM((EK,), jnp.int32),
        pltpu.VMEM((EK,), jnp.int32),
        pltpu.VMEM((EK, 128), jnp.float32),
        pltpu.VMEM((EK, 128), jnp.float32),
        pltpu.VMEM((EK, 128), jnp.float32),
        pltpu.VMEM((EK * L,), jnp.float32),
        pltpu.VMEM((EK * L,), jnp.float32),
        pltpu.VMEM((EK * L,), jnp.float32),
        pltpu.SemaphoreType.DMA,
        pltpu.SemaphoreType.DMA,
        pltpu.SemaphoreType.DMA,
        pltpu.SemaphoreType.DMA,
        pltpu.SemaphoreType.DMA,
        pltpu.SemaphoreType.DMA,
        pltpu.SemaphoreType.DMA,
        pltpu.SemaphoreType.DMA,
        pltpu.SemaphoreType.DMA,
        pltpu.VMEM_SHARED((NPAD, 128), jnp.float32),
    ]

    def body(feat_hbm, src_hbm, dst_hbm, g_hbm, z_hbm, out_hbm,
             srcf, dc0, dc1, dc2, rows0, rows1, rows2, gva, gvb, gvc,
             sr0, sr1, sr2, sg0, sg1, sg2, sd0, sd1, sd2, acc):
        c = lax.axis_index("c")
        s = lax.axis_index("s")
        rbase = s * ROWS_PER_TILE
        ebase = c * E_PER_SC + s * E_PER_TILE
        pltpu.sync_copy(z_hbm.at[pl.ds(rbase, ROWS_PER_TILE)],
                        acc.at[pl.ds(rbase, ROWS_PER_TILE)])
        pltpu.sync_copy(src_hbm.at[pl.ds(ebase, E_PER_TILE)], srcf)
        plsc.subcore_barrier()

        bufs = ((rows0, gva, dc0, sr0, sg0, sd0),
                (rows1, gvb, dc1, sr1, sg1, sd1),
                (rows2, gvc, dc2, sr2, sg2, sd2))

        def fire(ch, b):
            rows, gv, dc, sr, sg, sd = bufs[b]
            h1 = pltpu.async_copy(
                feat_hbm.at[srcf.at[pl.ds(ch * EK, EK)]], rows, sr)
            h3 = pltpu.async_copy(dst_hbm.at[pl.ds(ebase + ch * EK, EK)],
                                  dc, sd)
            if weighted:
                h2 = pltpu.async_copy(
                    g_hbm.at[pl.ds((ebase + ch * EK) * L, EK * L)], gv, sg)
                return h1, h2, h3
            return h1, h3

        def drain(ch, b, handles):
            rows, gv, dc, _, _, _ = bufs[b]
            for h in handles:
                h.wait()
            if weighted:
                def edge(e, _):
                    gv16 = gv[pl.ds(e * L, L)]
                    for j in range(128 // L):
                        rows[e, pl.ds(j * L, L)] = rows[e, pl.ds(j * L, L)] * gv16
                    return 0

                lax.fori_loop(0, EK, edge, 0)
            pltpu.sync_copy(rows, acc.at[dc], add=True)

        def triple(i, _):
            c0 = 3 * i
            h0 = fire(c0, 0)
            h1 = fire(c0 + 1, 1)
            h2 = fire(c0 + 2, 2)
            drain(c0, 0, h0)
            drain(c0 + 1, 1, h1)
            drain(c0 + 2, 2, h2)
            return 0

        lax.fori_loop(0, N_CHUNK // 3, triple, 0)
        for r in range(N_CHUNK % 3):
            ch = (N_CHUNK // 3) * 3 + r
            drain(ch, r, fire(ch, r))
        plsc.subcore_barrier()
        pltpu.sync_copy(acc.at[pl.ds(rbase, ROWS_PER_TILE)],
                        out_hbm.at[c, pl.ds(rbase, ROWS_PER_TILE)])

    return functools.partial(
        pl.kernel,
        out_type=jax.ShapeDtypeStruct((NC, NPAD, 128), jnp.float32),
        mesh=_MESH,
        scratch_types=scratch,
    )(body)


_edge_agg_plain = _make_edge_agg(False)
_edge_agg_w = _make_edge_agg(True)


BM = 1024
NBLK = NPAD // BM


def _gat_proj_body(x_ref, w_ref, alp_ref, arp_ref, feat_ref, el_ref, er_ref,
                   mx_ref):
    feat = x_ref[...] @ w_ref[...]
    feat_ref[...] = feat
    el = feat @ alp_ref[...]
    er = feat @ arp_ref[...]
    el_ref[...] = el
    er_ref[...] = er

    @pl.when(pl.program_id(0) == 0)
    def _():
        mx_ref[...] = jnp.full((8, 128), -1e30, jnp.float32)

    bmax = jnp.max(el, axis=0, keepdims=True)
    mx_ref[...] = jnp.maximum(mx_ref[...], jnp.broadcast_to(bmax, (8, 128)))


def _gat_proj(x, w, alp, arp):
    din, dout = w.shape
    return pl.pallas_call(
        _gat_proj_body,
        grid=(NBLK,),
        in_specs=[
            pl.BlockSpec((BM, din), lambda i: (i, 0)),
            pl.BlockSpec((din, dout), lambda i: (0, 0)),
            pl.BlockSpec((dout, 128), lambda i: (0, 0)),
            pl.BlockSpec((dout, 128), lambda i: (0, 0)),
        ],
        out_specs=[
            pl.BlockSpec((BM, dout), lambda i: (i, 0)),
            pl.BlockSpec((BM, 128), lambda i: (i, 0)),
            pl.BlockSpec((BM, 128), lambda i: (i, 0)),
            pl.BlockSpec((8, 128), lambda i: (0, 0)),
        ],
        out_shape=[
            jax.ShapeDtypeStruct((NPAD, dout), jnp.float32),
            jax.ShapeDtypeStruct((NPAD, 128), jnp.float32),
            jax.ShapeDtypeStruct((NPAD, 128), jnp.float32),
            jax.ShapeDtypeStruct((8, 128), jnp.float32),
        ],
    )(x, w, alp, arp)


def _make_gat_out_body(nhead, dout, act):
    def body(h_ref, r_ref, a0_ref, a1_ref, s0_ref, s1_ref, b_ref, out_ref):
        agg = a0_ref[...] + a1_ref[...]
        sden = s0_ref[...] + s1_ref[...]
        recip = 1.0 / (sden[:, :nhead] + 1e-9)
        if nhead > 1:
            rep = jnp.concatenate(
                [jnp.broadcast_to(recip[:, h:h + 1], (BM, dout))
                 for h in range(nhead)], axis=1)
        else:
            rep = jnp.broadcast_to(recip, (BM, dout))
        out = agg * rep + b_ref[0, :] + h_ref[...] @ r_ref[...]
        if act:
            out = jnp.maximum(out, 0.0)
        out_ref[...] = out
    return body


def _gat_out(h, r, a0, a1, s0, s1, b8, nhead, dout, act):
    din = h.shape[1]
    d = nhead * dout
    return pl.pallas_call(
        _make_gat_out_body(nhead, dout, act),
        grid=(NBLK,),
        in_specs=[
            pl.BlockSpec((BM, din), lambda i: (i, 0)),
            pl.BlockSpec((din, d), lambda i: (0, 0)),
            pl.BlockSpec((BM, d), lambda i: (i, 0)),
            pl.BlockSpec((BM, d), lambda i: (i, 0)),
            pl.BlockSpec((BM, 128), lambda i: (i, 0)),
            pl.BlockSpec((BM, 128), lambda i: (i, 0)),
            pl.BlockSpec((8, d), lambda i: (0, 0)),
        ],
        out_specs=pl.BlockSpec((BM, d), lambda i: (i, 0)),
        out_shape=jax.ShapeDtypeStruct((NPAD, d), jnp.float32),
    )(h, r, a0, a1, s0, s1, b8)


def _gin_mlp_body(h_ref, a0_ref, a1_ref, w1_ref, b1_ref, w2_ref, b2_ref,
                  out_ref):
    h = h_ref[...]
    t = h + a0_ref[...] + a1_ref[...]
    z = jnp.maximum(t @ w1_ref[...] + b1_ref[0, :], 0.0)
    out_ref[...] = z @ w2_ref[...] + b2_ref[0, :] + h


def _gin_mlp(h, a0, a1, w1, b18, w2, b28):
    return pl.pallas_call(
        _gin_mlp_body,
        grid=(NBLK,),
        in_specs=[
            pl.BlockSpec((BM, SUB), lambda i: (i, 0)),
            pl.BlockSpec((BM, SUB), lambda i: (i, 0)),
            pl.BlockSpec((BM, SUB), lambda i: (i, 0)),
            pl.BlockSpec((SUB, SUB), lambda i: (0, 0)),
            pl.BlockSpec((8, SUB), lambda i: (0, 0)),
            pl.BlockSpec((SUB, SUB), lambda i: (0, 0)),
            pl.BlockSpec((8, SUB), lambda i: (0, 0)),
        ],
        out_specs=pl.BlockSpec((BM, SUB), lambda i: (i, 0)),
        out_shape=jax.ShapeDtypeStruct((NPAD, SUB), jnp.float32),
    )(h, a0, a1, w1, b18, w2, b28)


def _pool_body(p_ref, x_ref, out_ref):
    @pl.when(pl.program_id(0) == 0)
    def _():
        out_ref[...] = jnp.zeros_like(out_ref)

    out_ref[...] += lax.dot_general(
        p_ref[...], x_ref[...], (((0,), (0,)), ((), ())),
        preferred_element_type=jnp.float32)


def _pool(p128, x):
    dx = x.shape[1]
    return pl.pallas_call(
        _pool_body,
        grid=(NBLK,),
        in_specs=[
            pl.BlockSpec((BM, 128), lambda i: (i, 0)),
            pl.BlockSpec((BM, dx), lambda i: (i, 0)),
        ],
        out_specs=pl.BlockSpec((128, dx), lambda i: (0, 0)),
        out_shape=jax.ShapeDtypeStruct((128, dx), jnp.float32),
    )(p128, x)


def _head_body(ps_ref, pf_ref, hm_ref, fpc_ref, adm_ref, admb_ref, adf_ref,
               adfb_ref, p1_ref, p2_ref, p3_ref, out_ref):
    pf = pf_ref[...]
    cnt = pf[:, SUB:SUB + 128]
    cnt2 = jnp.concatenate([cnt, cnt], axis=1)
    xf = pf[:, :SUB] / jnp.maximum(cnt2, 1.0)
    y = jnp.concatenate([ps_ref[...], xf], axis=1)
    y = y + hm_ref[...] @ adm_ref[...] + admb_ref[0, :]
    y = y + fpc_ref[...] @ adf_ref[...] + adfb_ref[0, :]
    z = jnp.maximum(y @ p1_ref[...], 0.0)
    z = jnp.maximum(z @ p2_ref[...], 0.0)
    out_ref[...] = z @ p3_ref[...]


def _head(ps, pf, hm, fpc, adm, admb8, adf, adfb8, p1, p2, p3p):
    return pl.pallas_call(
        _head_body,
        out_shape=jax.ShapeDtypeStruct((B, 128), jnp.float32),
    )(ps, pf, hm, fpc, adm, admb8, adf, adfb8, p1, p2, p3p)


def _row8(v):
    return jnp.broadcast_to(v[None, :], (8, v.shape[0]))


def _blockdiag_attn(a, dout):
    nh = a.shape[0]
    rows = jnp.arange(nh * dout)
    return jnp.where(jnp.arange(128)[None, :] == (rows // dout)[:, None],
                     a.reshape(-1)[:, None], 0.0).astype(jnp.float32)


def _gat_layer(h, src4, dst4, W, al, ar, b, R, nhead, dout, act, zeros128):
    d = nhead * dout
    alp = _blockdiag_attn(al, dout)
    arp = _blockdiag_attn(ar, dout)
    feat, el128, er128, mx = _gat_proj(h, W, alp, arp)
    lane_pad = jnp.where(jnp.arange(L) < nhead, 0.0, -1e30).astype(jnp.float32)
    el16 = el128[:, :L] + lane_pad[None, :]
    cpre = mx[0:1, :L] + er128[:, :L]
    c16 = jnp.where(cpre > 0, cpre, cpre * SLOPE)
    er16 = er128[:, :L]
    tcomb = jnp.concatenate(
        [el16, er16, c16, jnp.zeros((NPAD, 128 - 3 * L), jnp.float32)], axis=1)

    gf, s_part = _sc_edge_gate(tcomb, src4, dst4, zeros128)
    g = gf.reshape(E, L)
    s0 = s_part[0]
    s1 = s_part[1]

    nslice = d // 128
    featperm = feat.reshape(NPAD, nslice, 128).transpose(1, 0, 2)
    parts0 = []
    parts1 = []
    for p in range(nslice):
        lane = p if nhead > 1 else 0
        gx = jnp.broadcast_to(g[:, lane:lane + 1], (E, L)).reshape(E * L)
        part = _edge_agg_w(featperm[p], src4, dst4, gx, zeros128)
        parts0.append(part[0])
        parts1.append(part[1])
    a0 = jnp.concatenate(parts0, axis=1)
    a1 = jnp.concatenate(parts1, axis=1)

    return _gat_out(h, R, a0, a1, s0, s1, _row8(b), nhead, dout, act)


def _gin_layer(h, src4, dst4, w1, b1, w2, b2, zeros128, g_dummy):
    hperm = h.reshape(NPAD, 2, 128).transpose(1, 0, 2)
    pa = _edge_agg_plain(hperm[0], src4, dst4, g_dummy, zeros128)
    pb = _edge_agg_plain(hperm[1], src4, dst4, g_dummy, zeros128)
    a0 = jnp.concatenate([pa[0], pb[0]], axis=1)
    a1 = jnp.concatenate([pa[1], pb[1]], axis=1)
    return _gin_mlp(h, a0, a1, w1, _row8(b1), w2, _row8(b2))


def kernel(node_feature, h_MolCLR, maccs, morgan, params, edge_index,
           node_subgraph, graph_ids):
    p = params
    src4 = edge_index[0].astype(jnp.int32)
    dst4 = edge_index[1].astype(jnp.int32)

    zeros128 = jnp.zeros((NPAD, 128), jnp.float32)
    g_dummy = jnp.zeros((E * L,), jnp.float32)

    idx_pad = jnp.zeros((NPAD,), jnp.int32).at[:N].set(
        node_subgraph.astype(jnp.int32))
    x_sub = _sc_gather(p['embed'], idx_pad)

    x_sub = _gin_layer(x_sub, src4, dst4, p['gin_w1_0'], p['gin_b1_0'],
                       p['gin_w2_0'], p['gin_b2_0'], zeros128, g_dummy)
    x_sub = _gin_layer(x_sub, src4, dst4, p['gin_w1_1'], p['gin_b1_1'],
                       p['gin_w2_1'], p['gin_b2_1'], zeros128, g_dummy)

    gid_pad = jnp.full((NPAD,), -1, jnp.int32).at[:N].set(
        graph_ids.astype(jnp.int32))
    p128 = (gid_pad[:, None] == jnp.arange(128)[None, :]).astype(jnp.float32)

    pool_sub = _pool(p128, x_sub)[:B, :]

    hf = jnp.zeros((NPAD, FEAT), jnp.float32).at[:N].set(node_feature)
    x_feat = _gat_layer(hf, src4, dst4, p['gat_W_0'],
                        p['gat_al_0'], p['gat_ar_0'], p['gat_b_0'],
                        p['gat_R_0'], NH1, H1, True, zeros128)
    x_feat = _gat_layer(x_feat, src4, dst4, p['gat_W_1'],
                        p['gat_al_1'], p['gat_ar_1'], p['gat_b_1'],
                        p['gat_R_1'], NH2, O2, False, zeros128)

    xc = jnp.concatenate([x_feat, jnp.where(gid_pad >= 0, 1.0, 0.0)[:, None]
                          * jnp.ones((NPAD, 128), jnp.float32)], axis=1)
    pool_feat = _pool(p128, xc)[:B, :]

    fpc = jnp.zeros((B, 384), jnp.float32)
    fpc = fpc.at[:, :167].set(maccs.astype(jnp.float32))
    fpc = fpc.at[:, 167:367].set(morgan.astype(jnp.float32))
    adf = jnp.zeros((384, MOL), jnp.float32).at[:367].set(p['adF_w'])
    p3p = jnp.zeros((128, 128), jnp.float32).at[:, :11].set(p['p3'])

    out = _head(pool_sub, pool_feat, h_MolCLR, fpc,
                p['adM_w'], _row8(p['adM_b']), adf, _row8(p['adF_b']),
                p['p1'], p['p2'], p3p)
    return out[:, :11]
